# baseline (speedup 1.0000x reference)
import functools
import math

import jax
import jax.numpy as jnp
from jax import lax
from jax.experimental import pallas as pl
from jax.experimental.pallas import tpu as pltpu
from jax.experimental.pallas import tpu_sc as plsc

_B = 1024
_V = 100000
_SMOOTHING = 0.1
_CONFIDENCE = 1.0 - _SMOOTHING
_FILL = _SMOOTHING / (_V - 2)
_DELTA = _CONFIDENCE - _FILL
_CONST = _B * ((_V - 1) * _FILL * math.log(_FILL)
               + _CONFIDENCE * math.log(_CONFIDENCE))

_NC, _NS, _L = 2, 16, 16
_NW = _NC * _NS
_BPW = _B // _NW

_SCV = 40000
_CHR = 40
_NCH = _SCV // _CHR
_CPW = 31

_sc_mesh = plsc.VectorSubcoreMesh(core_axis_name="c", subcore_axis_name="s")


def _sum_chunk(buf, accs):
    def row(r, a):
        res = list(a)
        for g in range(_B // _L // 8):
            for u in range(8):
                res[u] = res[u] + buf[r, pl.ds((g * 8 + u) * _L, _L)]
        return tuple(res)
    return list(plsc.parallel_loop(0, _CHR, carry=tuple(accs))(row))


@functools.partial(
    pl.kernel,
    out_type=jax.ShapeDtypeStruct((2 * _NW * _L,), jnp.float32),
    mesh=_sc_mesh,
    compiler_params=pltpu.CompilerParams(needs_layout_passes=False),
    scratch_types=[
        pltpu.VMEM((_BPW,), jnp.int32),
        pltpu.VMEM((_BPW, 8, 128), jnp.float32),
        pltpu.VMEM((_CHR, _B), jnp.float32),
        pltpu.VMEM((_CHR, _B), jnp.float32),
        pltpu.VMEM((_L,), jnp.float32),
        pltpu.VMEM((_L,), jnp.float32),
        pltpu.SemaphoreType.DMA,
        pltpu.SemaphoreType.DMA,
        pltpu.SemaphoreType.DMA,
    ],
)
def _sc_part(xt_hbm, tgt_hbm, out_hbm, tgt_v, win_v, buf0, buf1,
             acc_v, sum_v, gsem, sem0, sem1):
    wid = lax.axis_index("s") * _NC + lax.axis_index("c")
    base = wid * _BPW
    lanes = lax.broadcasted_iota(jnp.int32, (_L,), 0)

    pltpu.sync_copy(tgt_hbm.at[pl.ds(base, _BPW)], tgt_v)
    ts = []
    copies = []
    for c in range(_BPW // _L):
        tvec = tgt_v[pl.ds(c * _L, _L)]
        for l in range(_L):
            k = c * _L + l
            t = jnp.sum(jnp.where(lanes == l, tvec, 0))
            ts.append(t)
            row8 = (t >> 3) * 8
            col128 = ((base + k) >> 7) * 128
            copies.append(pltpu.async_copy(
                xt_hbm.at[pl.ds(row8, 8), pl.ds(col128, 128)],
                win_v.at[k], gsem))
    for cp in copies:
        cp.wait()

    acc = jnp.zeros((_L,), jnp.float32)
    for k in range(_BPW):
        t = ts[k]
        sub = (base & 127) + (k & -_L)
        w = win_v[k, t & 7, pl.ds(sub, _L)]
        acc = acc + jnp.where(lanes == (k & (_L - 1)), w, 0.0)
    acc_v[...] = acc
    pltpu.sync_copy(acc_v, out_hbm.at[pl.ds(wid * _L, _L)])

    def chunk_src(i):
        return xt_hbm.at[pl.ds((wid + i * _NW) * _CHR, _CHR), pl.ds(0, _B)]

    accs = [jnp.zeros((_L,), jnp.float32) for _ in range(8)]
    pltpu.async_copy(chunk_src(0), buf0, sem0)

    def pair(c, accs_t):
        accs_l = list(accs_t)
        pltpu.make_async_copy(chunk_src(2 * c), buf0, sem0).wait()
        pltpu.async_copy(chunk_src(2 * c + 1), buf1, sem1)
        accs_l = _sum_chunk(buf0, accs_l)
        pltpu.make_async_copy(chunk_src(2 * c + 1), buf1, sem1).wait()
        pltpu.async_copy(chunk_src(2 * c + 2), buf0, sem0)
        accs_l = _sum_chunk(buf1, accs_l)
        return tuple(accs_l)

    accs = list(lax.fori_loop(0, (_CPW - 1) // 2, pair, tuple(accs)))
    pltpu.make_async_copy(chunk_src(_CPW - 1), buf0, sem0).wait()
    accs = _sum_chunk(buf0, accs)

    @pl.when(wid < _NCH - _CPW * _NW)
    def _extra():
        extra = pltpu.async_copy(
            xt_hbm.at[pl.ds((_CPW * _NW + wid) * _CHR, _CHR), pl.ds(0, _B)],
            buf1, sem1)
        extra.wait()
        a2 = _sum_chunk(buf1, [jnp.zeros((_L,), jnp.float32)] * 8)
        s2 = a2[0]
        for a in a2[1:]:
            s2 = s2 + a
        sum_v[...] = s2

    @pl.when(wid >= _NCH - _CPW * _NW)
    def _noextra():
        sum_v[...] = jnp.zeros((_L,), jnp.float32)

    s = accs[0]
    for a in accs[1:]:
        s = s + a
    sum_v[...] = sum_v[...] + s
    pltpu.sync_copy(sum_v, out_hbm.at[pl.ds(_NW * _L + wid * _L, _L)])


_W0 = 2000
_TCOFF = _SCV // _W0
_GRID = (_V - _SCV) // _W0


def _sum_body(xa_ref, xb_ref, xc_ref, xd_ref, out_ref, acc_ref):
    j = pl.program_id(0)

    @pl.when(j == 0)
    def _init():
        acc_ref[0] = 0.0

    acc_ref[0] += jnp.sum((xa_ref[...] + xb_ref[...]) + (xc_ref[...] + xd_ref[...]))

    @pl.when(j == _GRID - 1)
    def _last():
        out_ref[0, 0] = acc_ref[0]


def _combine_body(g_ref, t_ref, out_ref):
    s2 = jnp.sum(g_ref[0:4, :])
    s1 = t_ref[0, 0] + jnp.sum(g_ref[4:8, :])
    loss = _CONST - _FILL * s1 - _DELTA * s2
    out_ref[0, 0] = loss.astype(jnp.float32)


@jax.jit
def kernel(x, target):
    xt = x.T
    g = _sc_part(xt, target)
    tc = pl.pallas_call(
        _sum_body,
        grid=(_GRID,),
        in_specs=[pl.BlockSpec((_W0, _B // 4), (lambda i: lambda j: (j + _TCOFF, i))(i))
                  for i in range(4)],
        out_specs=pl.BlockSpec(memory_space=pltpu.SMEM),
        out_shape=jax.ShapeDtypeStruct((1, 1), jnp.float32),
        scratch_shapes=[pltpu.SMEM((1,), jnp.float32)],
    )(xt, xt, xt, xt)
    out = pl.pallas_call(
        _combine_body,
        in_specs=[
            pl.BlockSpec((8, 128), lambda: (0, 0)),
            pl.BlockSpec(memory_space=pltpu.SMEM),
        ],
        out_specs=pl.BlockSpec(memory_space=pltpu.SMEM),
        out_shape=jax.ShapeDtypeStruct((1, 1), jnp.float32),
    )(g.reshape(8, 128), tc)
    return out[0, 0]


# speedup vs baseline: 1.0028x; 1.0028x over previous
import functools
import math

import jax
import jax.numpy as jnp
from jax import lax
from jax.experimental import pallas as pl
from jax.experimental.pallas import tpu as pltpu
from jax.experimental.pallas import tpu_sc as plsc

_B = 1024
_V = 100000
_SMOOTHING = 0.1
_CONFIDENCE = 1.0 - _SMOOTHING
_FILL = _SMOOTHING / (_V - 2)
_DELTA = _CONFIDENCE - _FILL
_CONST = _B * ((_V - 1) * _FILL * math.log(_FILL)
               + _CONFIDENCE * math.log(_CONFIDENCE))

_NC, _NS, _L = 2, 16, 16
_NW = _NC * _NS
_BPW = _B // _NW

_SCV = 40000
_CHR = 40
_NCH = _SCV // _CHR
_CPW = 31

_sc_mesh = plsc.VectorSubcoreMesh(core_axis_name="c", subcore_axis_name="s")


def _sum_chunk(buf, accs):
    def row(r, a):
        res = list(a)
        for g in range(_B // _L // 8):
            for u in range(8):
                res[u] = res[u] + buf[r, pl.ds((g * 8 + u) * _L, _L)]
        return tuple(res)
    return list(lax.fori_loop(0, _CHR, row, tuple(accs)))


@functools.partial(
    pl.kernel,
    out_type=jax.ShapeDtypeStruct((2 * _NW * _L,), jnp.float32),
    mesh=_sc_mesh,
    compiler_params=pltpu.CompilerParams(needs_layout_passes=False),
    scratch_types=[
        pltpu.VMEM((_BPW,), jnp.int32),
        pltpu.VMEM((_BPW, 8, 128), jnp.float32),
        pltpu.VMEM((_CHR, _B), jnp.float32),
        pltpu.VMEM((_CHR, _B), jnp.float32),
        pltpu.VMEM((_L,), jnp.float32),
        pltpu.VMEM((_L,), jnp.float32),
        pltpu.SemaphoreType.DMA,
        pltpu.SemaphoreType.DMA,
        pltpu.SemaphoreType.DMA,
    ],
)
def _sc_part(xt_hbm, tgt_hbm, out_hbm, tgt_v, win_v, buf0, buf1,
             acc_v, sum_v, gsem, sem0, sem1):
    wid = lax.axis_index("s") * _NC + lax.axis_index("c")
    base = wid * _BPW
    lanes = lax.broadcasted_iota(jnp.int32, (_L,), 0)

    pltpu.sync_copy(tgt_hbm.at[pl.ds(base, _BPW)], tgt_v)
    ts = []
    copies = []
    for c in range(_BPW // _L):
        tvec = tgt_v[pl.ds(c * _L, _L)]
        for l in range(_L):
            k = c * _L + l
            t = jnp.sum(jnp.where(lanes == l, tvec, 0))
            ts.append(t)
            row8 = (t >> 3) * 8
            col128 = ((base + k) >> 7) * 128
            copies.append(pltpu.async_copy(
                xt_hbm.at[pl.ds(row8, 8), pl.ds(col128, 128)],
                win_v.at[k], gsem))
    for cp in copies:
        cp.wait()

    acc = jnp.zeros((_L,), jnp.float32)
    for k in range(_BPW):
        t = ts[k]
        sub = (base & 127) + (k & -_L)
        w = win_v[k, t & 7, pl.ds(sub, _L)]
        acc = acc + jnp.where(lanes == (k & (_L - 1)), w, 0.0)
    acc_v[...] = acc
    pltpu.sync_copy(acc_v, out_hbm.at[pl.ds(wid * _L, _L)])

    def chunk_src(i):
        return xt_hbm.at[pl.ds((wid + i * _NW) * _CHR, _CHR), pl.ds(0, _B)]

    accs = [jnp.zeros((_L,), jnp.float32) for _ in range(8)]
    pltpu.async_copy(chunk_src(0), buf0, sem0)

    def pair(c, accs_t):
        accs_l = list(accs_t)
        pltpu.make_async_copy(chunk_src(2 * c), buf0, sem0).wait()
        pltpu.async_copy(chunk_src(2 * c + 1), buf1, sem1)
        accs_l = _sum_chunk(buf0, accs_l)
        pltpu.make_async_copy(chunk_src(2 * c + 1), buf1, sem1).wait()
        pltpu.async_copy(chunk_src(2 * c + 2), buf0, sem0)
        accs_l = _sum_chunk(buf1, accs_l)
        return tuple(accs_l)

    accs = list(lax.fori_loop(0, (_CPW - 1) // 2, pair, tuple(accs)))
    pltpu.make_async_copy(chunk_src(_CPW - 1), buf0, sem0).wait()
    accs = _sum_chunk(buf0, accs)

    @pl.when(wid < _NCH - _CPW * _NW)
    def _extra():
        extra = pltpu.async_copy(
            xt_hbm.at[pl.ds((_CPW * _NW + wid) * _CHR, _CHR), pl.ds(0, _B)],
            buf1, sem1)
        extra.wait()
        a2 = _sum_chunk(buf1, [jnp.zeros((_L,), jnp.float32)] * 8)
        s2 = a2[0]
        for a in a2[1:]:
            s2 = s2 + a
        sum_v[...] = s2

    @pl.when(wid >= _NCH - _CPW * _NW)
    def _noextra():
        sum_v[...] = jnp.zeros((_L,), jnp.float32)

    s = accs[0]
    for a in accs[1:]:
        s = s + a
    sum_v[...] = sum_v[...] + s
    pltpu.sync_copy(sum_v, out_hbm.at[pl.ds(_NW * _L + wid * _L, _L)])


_W0 = 2000
_TCOFF = _SCV // _W0
_GRID = (_V - _SCV) // _W0


def _sum_body(xa_ref, xb_ref, xc_ref, xd_ref, out_ref, acc_ref):
    j = pl.program_id(0)

    @pl.when(j == 0)
    def _init():
        acc_ref[0] = 0.0

    acc_ref[0] += jnp.sum((xa_ref[...] + xb_ref[...]) + (xc_ref[...] + xd_ref[...]))

    @pl.when(j == _GRID - 1)
    def _last():
        out_ref[0, 0] = acc_ref[0]


def _combine_body(g_ref, t_ref, out_ref):
    s2 = jnp.sum(g_ref[0:4, :])
    s1 = t_ref[0, 0] + jnp.sum(g_ref[4:8, :])
    loss = _CONST - _FILL * s1 - _DELTA * s2
    out_ref[0, 0] = loss.astype(jnp.float32)


@jax.jit
def kernel(x, target):
    xt = x.T
    g = _sc_part(xt, target)
    tc = pl.pallas_call(
        _sum_body,
        grid=(_GRID,),
        in_specs=[pl.BlockSpec((_W0, _B // 4), (lambda i: lambda j: (j + _TCOFF, i))(i))
                  for i in range(4)],
        out_specs=pl.BlockSpec(memory_space=pltpu.SMEM),
        out_shape=jax.ShapeDtypeStruct((1, 1), jnp.float32),
        scratch_shapes=[pltpu.SMEM((1,), jnp.float32)],
    )(xt, xt, xt, xt)
    out = pl.pallas_call(
        _combine_body,
        in_specs=[
            pl.BlockSpec((8, 128), lambda: (0, 0)),
            pl.BlockSpec(memory_space=pltpu.SMEM),
        ],
        out_specs=pl.BlockSpec(memory_space=pltpu.SMEM),
        out_shape=jax.ShapeDtypeStruct((1, 1), jnp.float32),
    )(g.reshape(8, 128), tc)
    return out[0, 0]
